# Initial kernel scaffold; baseline (speedup 1.0000x reference)
import jax
import jax.numpy as jnp
from jax.experimental import pallas as pl


def kernel(tokens, tok_table, pos_table, gamma, beta):
    raise NotImplementedError("write your pallas kernel here")


# speedup vs baseline: 1.1597x; 1.1597x over previous
import functools

import jax
import jax.numpy as jnp
from jax import lax
from jax.experimental import pallas as pl
from jax.experimental.pallas import tpu as pltpu
from jax.experimental.pallas import tpu_sc as plsc

_D = 64
_SCALE = 8.0
_EPS = 1e-5
_POS_PAD = 256
_GATHER_W = 128
_LN_ROWS = 2048
_POS_BLK = 128


def _sc_gather(table, idx):
    n = idx.shape[1]
    d = table.shape[1]
    mesh = plsc.VectorSubcoreMesh(core_axis_name="c", subcore_axis_name="s")

    @functools.partial(
        pl.kernel,
        out_type=jax.ShapeDtypeStruct((n, d), table.dtype),
        mesh=mesh,
        compiler_params=pltpu.CompilerParams(use_tc_tiling_on_sc=False),
    )
    def gather_k(tab_hbm, idx_hbm, out_hbm):
        def body(i_vmem, o_vmem):
            pltpu.sync_copy(tab_hbm.at[i_vmem.at[0]], o_vmem)

        pltpu.emit_pipeline(
            body,
            grid=(n // _GATHER_W,),
            in_specs=[pl.BlockSpec((1, _GATHER_W), lambda i: (0, i))],
            out_specs=[pl.BlockSpec((_GATHER_W, d), lambda i: (i, 0))],
            core_axis_name=("c", "s"),
            dimension_semantics=(pltpu.PARALLEL,),
        )(idx_hbm, out_hbm)

    return gather_k(table, idx)


def _positions_body(tok_ref, pos_ref):
    tok = tok_ref[...]
    mask = tok != 0
    mb = mask.astype(jnp.bfloat16)
    s = tok.shape[1]
    r = lax.broadcasted_iota(jnp.int32, (s, s), 0)
    c = lax.broadcasted_iota(jnp.int32, (s, s), 1)
    tri = (r <= c).astype(jnp.bfloat16)
    cs = jnp.dot(mb, tri, preferred_element_type=jnp.float32)
    pos_ref[...] = cs * mask.astype(jnp.float32)


def _ln_body(tok_ref, pos_ref, hi_ref, lo_ref, gamma_ref, beta_ref, out_ref):
    x = tok_ref[...] * _SCALE
    posi = pos_ref[...].astype(jnp.int32)
    io = lax.broadcasted_iota(jnp.int32, (posi.shape[0], _POS_PAD), 1)
    oh = (posi == io).astype(jnp.bfloat16)
    pe = jnp.dot(oh, hi_ref[...], preferred_element_type=jnp.float32)
    pe = pe + jnp.dot(oh, lo_ref[...], preferred_element_type=jnp.float32)
    x = x + pe
    mean = jnp.mean(x, axis=1, keepdims=True)
    xc = x - mean
    var = jnp.mean(xc * xc, axis=1, keepdims=True)
    inv = lax.rsqrt(var + _EPS)
    out_ref[...] = xc * inv * gamma_ref[...] + beta_ref[...]


def kernel(tokens, tok_table, pos_table, gamma, beta):
    b, s = tokens.shape
    d = tok_table.shape[1]
    n = b * s
    tokens = tokens.astype(jnp.int32)

    tok_emb = _sc_gather(tok_table, tokens.reshape(1, n))

    pos = pl.pallas_call(
        _positions_body,
        grid=(b // _POS_BLK,),
        in_specs=[pl.BlockSpec((_POS_BLK, s), lambda i: (i, 0))],
        out_specs=pl.BlockSpec((_POS_BLK, s), lambda i: (i, 0)),
        out_shape=jax.ShapeDtypeStruct((b, s), jnp.float32),
    )(tokens)

    pt = jnp.zeros((_POS_PAD, d), jnp.float32).at[: pos_table.shape[0]].set(pos_table)
    hi = pt.astype(jnp.bfloat16)
    lo = (pt - hi.astype(jnp.float32)).astype(jnp.bfloat16)

    out = pl.pallas_call(
        _ln_body,
        grid=(n // _LN_ROWS,),
        in_specs=[
            pl.BlockSpec((_LN_ROWS, d), lambda i: (i, 0)),
            pl.BlockSpec((_LN_ROWS, 1), lambda i: (i, 0)),
            pl.BlockSpec((_POS_PAD, d), lambda i: (0, 0)),
            pl.BlockSpec((_POS_PAD, d), lambda i: (0, 0)),
            pl.BlockSpec((1, d), lambda i: (0, 0)),
            pl.BlockSpec((1, d), lambda i: (0, 0)),
        ],
        out_specs=pl.BlockSpec((_LN_ROWS, d), lambda i: (i, 0)),
        out_shape=jax.ShapeDtypeStruct((n, d), jnp.float32),
    )(tok_emb, pos.reshape(n, 1), hi, lo, gamma.reshape(1, d), beta.reshape(1, d))

    return out.reshape(b, s, d)
